# Initial kernel scaffold; baseline (speedup 1.0000x reference)
import functools

import jax
import jax.numpy as jnp
from jax import lax
from jax.experimental import pallas as pl
from jax.experimental.pallas import tpu as pltpu
from jax.experimental.pallas import tpu_sc as plsc

N0 = 266240
N1 = 10240
N2 = 1024
E0 = 256000
E1 = 10240
F = 128
C = 32
DW = 16

NC = 2
NS = 16
NW = NC * NS

CH0, C0 = 64, 125
CH1, C1 = 4, 80

_mesh = plsc.VectorSubcoreMesh(core_axis_name="c", subcore_axis_name="s")


def _sc_aggregate(n_dst, row_w, n_chunks, chunk):
  rpt = n_dst // NS

  @functools.partial(
      pl.kernel,
      out_type=(
          jax.ShapeDtypeStruct((NC, n_dst, row_w), jnp.float32),
          jax.ShapeDtypeStruct((NC, n_dst, DW), jnp.float32),
      ),
      mesh=_mesh,
      scratch_types=[
          pltpu.VMEM((n_chunks, chunk), jnp.int32),
          pltpu.VMEM((n_chunks, chunk), jnp.int32),
          pltpu.VMEM((chunk, row_w), jnp.float32),
          pltpu.VMEM((chunk, DW), jnp.float32),
          pltpu.VMEM_SHARED((n_dst, row_w), jnp.float32),
          pltpu.VMEM_SHARED((n_dst, DW), jnp.float32),
          pltpu.SemaphoreType.DMA,
      ],
  )
  def agg(table_hbm, es_hbm, ed_hbm, zacc_hbm, zdeg_hbm, ones_hbm,
          acc_out, deg_out,
          idx_s, idx_d, rows, ones_v, acc_sh, deg_sh, sem):
    cid = lax.axis_index("c")
    sid = lax.axis_index("s")
    wid = sid * NC + cid
    my = pl.ds(sid * rpt, rpt)
    pltpu.sync_copy(zacc_hbm.at[my], acc_sh.at[my])
    pltpu.sync_copy(zdeg_hbm.at[my], deg_sh.at[my])
    pltpu.sync_copy(es_hbm.at[wid], idx_s)
    pltpu.sync_copy(ed_hbm.at[wid], idx_d)
    pltpu.sync_copy(ones_hbm, ones_v)
    plsc.subcore_barrier()

    def body(j, carry):
      pltpu.async_copy(table_hbm.at[idx_s.at[j]], rows, sem).wait()
      pltpu.sync_copy(rows, acc_sh.at[idx_d.at[j]], add=True)
      pltpu.sync_copy(ones_v, deg_sh.at[idx_d.at[j]], add=True)
      return carry

    lax.fori_loop(0, n_chunks, body, 0)
    plsc.subcore_barrier()
    pltpu.sync_copy(acc_sh.at[my], acc_out.at[cid, my])
    pltpu.sync_copy(deg_sh.at[my], deg_out.at[cid, my])

  return agg


_sc_agg0 = _sc_aggregate(N1, F, CH0, C0)
_sc_agg1 = _sc_aggregate(N2, C, CH1, C1)


def _tc1_body(x_ref, acc_ref, deg_ref, ws_ref, wn_ref, b_ref, wn2_ref,
              h_ref, p_ref):
  a = acc_ref[...]
  dg = deg_ref[...]
  d = (dg[0] + dg[1])[:, 0:1]
  inv = 1.0 / jnp.maximum(d, 1.0)
  hn = (a[0] + a[1]) * inv
  h = (jnp.dot(x_ref[...], ws_ref[...], preferred_element_type=jnp.float32)
       + jnp.dot(hn, wn_ref[...], preferred_element_type=jnp.float32)
       + b_ref[...])
  h = jnp.maximum(h, 0.0)
  h_ref[...] = h
  p_ref[...] = jnp.dot(h, wn2_ref[...], preferred_element_type=jnp.float32)


def _tc2_body(h_ref, acc_ref, deg_ref, ws_ref, b_ref, o_ref):
  a = acc_ref[...]
  dg = deg_ref[...]
  d = (dg[0] + dg[1])[:, 0:1]
  inv = 1.0 / jnp.maximum(d, 1.0)
  o_ref[...] = (jnp.dot(h_ref[...], ws_ref[...],
                        preferred_element_type=jnp.float32)
                + (a[0] + a[1]) * inv + b_ref[...])


_BT = 1024


def kernel(x, edge_src0, edge_dst0, edge_src1, edge_dst1,
           W_self1, W_neigh1, b1, W_self2, W_neigh2, b2):
  es0 = edge_src0.reshape(NW, CH0, C0)
  ed0 = edge_dst0.reshape(NW, CH0, C0)
  es1 = edge_src1.reshape(NW, CH1, C1)
  ed1 = edge_dst1.reshape(NW, CH1, C1)

  zacc0 = jnp.zeros((N1, F), jnp.float32)
  zdeg0 = jnp.zeros((N1, DW), jnp.float32)
  ones0 = jnp.ones((C0, DW), jnp.float32)
  acc, deg = _sc_agg0(x, es0, ed0, zacc0, zdeg0, ones0)

  h, p = pl.pallas_call(
      _tc1_body,
      grid=(N1 // _BT,),
      in_specs=[
          pl.BlockSpec((_BT, F), lambda i: (i, 0)),
          pl.BlockSpec((NC, _BT, F), lambda i: (0, i, 0)),
          pl.BlockSpec((NC, _BT, DW), lambda i: (0, i, 0)),
          pl.BlockSpec((F, F), lambda i: (0, 0)),
          pl.BlockSpec((F, F), lambda i: (0, 0)),
          pl.BlockSpec((1, F), lambda i: (0, 0)),
          pl.BlockSpec((F, C), lambda i: (0, 0)),
      ],
      out_specs=[
          pl.BlockSpec((_BT, F), lambda i: (i, 0)),
          pl.BlockSpec((_BT, C), lambda i: (i, 0)),
      ],
      out_shape=[
          jax.ShapeDtypeStruct((N1, F), jnp.float32),
          jax.ShapeDtypeStruct((N1, C), jnp.float32),
      ],
  )(x, acc, deg, W_self1, W_neigh1, b1.reshape(1, F), W_neigh2)

  zacc1 = jnp.zeros((N2, C), jnp.float32)
  zdeg1 = jnp.zeros((N2, DW), jnp.float32)
  ones1 = jnp.ones((C1, DW), jnp.float32)
  acc2, deg2 = _sc_agg1(p, es1, ed1, zacc1, zdeg1, ones1)

  out = pl.pallas_call(
      _tc2_body,
      grid=(1,),
      in_specs=[
          pl.BlockSpec((N2, F), lambda i: (0, 0)),
          pl.BlockSpec((NC, N2, C), lambda i: (0, 0, 0)),
          pl.BlockSpec((NC, N2, DW), lambda i: (0, 0, 0)),
          pl.BlockSpec((F, C), lambda i: (0, 0)),
          pl.BlockSpec((1, C), lambda i: (0, 0)),
      ],
      out_specs=pl.BlockSpec((N2, C), lambda i: (i, 0)),
      out_shape=jax.ShapeDtypeStruct((N2, C), jnp.float32),
  )(h, acc2, deg2, W_self2, b2.reshape(1, C))
  return out


# speedup vs baseline: 4.3067x; 4.3067x over previous
import functools

import jax
import jax.numpy as jnp
from jax import lax
from jax.experimental import pallas as pl
from jax.experimental.pallas import tpu as pltpu
from jax.experimental.pallas import tpu_sc as plsc

N0 = 266240
N1 = 10240
N2 = 1024
E0 = 256000
E1 = 10240
F = 128
C = 32
DW = 16

NC = 2
NS = 16
NW = NC * NS

CH0, C0 = 64, 125
CH1, C1 = 4, 80

_mesh = plsc.VectorSubcoreMesh(core_axis_name="c", subcore_axis_name="s")


@functools.partial(
    pl.kernel,
    out_type=(
        jax.ShapeDtypeStruct((NC, N1, F), jnp.float32),
        jax.ShapeDtypeStruct((NC, N1, F), jnp.float32),
    ),
    mesh=_mesh,
    scratch_types=[
        pltpu.VMEM((CH0, C0), jnp.int32),
        pltpu.VMEM((CH0, C0), jnp.int32),
        pltpu.VMEM((C0, F), jnp.float32),
        pltpu.VMEM((C0, F), jnp.float32),
        pltpu.VMEM_SHARED((N1, F), jnp.float32),
        pltpu.SemaphoreType.DMA,
    ],
)
def _sc_agg0(table_hbm, es_hbm, ed_hbm, zacc_hbm, ones_hbm,
             acc_out, deg_out,
             idx_s, idx_d, rows, ones_v, acc_sh, sem):
  cid = lax.axis_index("c")
  sid = lax.axis_index("s")
  wid = sid * NC + cid
  rpt = N1 // NS
  my = pl.ds(sid * rpt, rpt)
  pltpu.sync_copy(zacc_hbm.at[my], acc_sh.at[my])
  pltpu.sync_copy(es_hbm.at[wid], idx_s)
  pltpu.sync_copy(ed_hbm.at[wid], idx_d)
  pltpu.sync_copy(ones_hbm, ones_v)
  plsc.subcore_barrier()

  def body(j, carry):
    pltpu.async_copy(table_hbm.at[idx_s.at[j]], rows, sem).wait()
    pltpu.sync_copy(rows, acc_sh.at[idx_d.at[j]], add=True)
    return carry

  lax.fori_loop(0, CH0, body, 0)
  plsc.subcore_barrier()
  pltpu.sync_copy(acc_sh.at[my], acc_out.at[cid, my])
  plsc.subcore_barrier()
  pltpu.sync_copy(zacc_hbm.at[my], acc_sh.at[my])
  plsc.subcore_barrier()

  def body_deg(j, carry):
    pltpu.sync_copy(ones_v, acc_sh.at[idx_d.at[j]], add=True)
    return carry

  lax.fori_loop(0, CH0, body_deg, 0)
  plsc.subcore_barrier()
  pltpu.sync_copy(acc_sh.at[my], deg_out.at[cid, my])


@functools.partial(
    pl.kernel,
    out_type=(
        jax.ShapeDtypeStruct((NC, N2, F), jnp.float32),
        jax.ShapeDtypeStruct((NC, N2, F), jnp.float32),
    ),
    mesh=_mesh,
    scratch_types=[
        pltpu.VMEM((CH1, C1), jnp.int32),
        pltpu.VMEM((CH1, C1), jnp.int32),
        pltpu.VMEM((C1, F), jnp.float32),
        pltpu.VMEM((C1, F), jnp.float32),
        pltpu.VMEM_SHARED((N2, F), jnp.float32),
        pltpu.VMEM_SHARED((N2, F), jnp.float32),
        pltpu.SemaphoreType.DMA,
    ],
)
def _sc_agg1(table_hbm, es_hbm, ed_hbm, zacc_hbm, ones_hbm,
             acc_out, deg_out,
             idx_s, idx_d, rows, ones_v, acc_sh, deg_sh, sem):
  cid = lax.axis_index("c")
  sid = lax.axis_index("s")
  wid = sid * NC + cid
  rpt = N2 // NS
  my = pl.ds(sid * rpt, rpt)
  pltpu.sync_copy(zacc_hbm.at[my], acc_sh.at[my])
  pltpu.sync_copy(zacc_hbm.at[my], deg_sh.at[my])
  pltpu.sync_copy(es_hbm.at[wid], idx_s)
  pltpu.sync_copy(ed_hbm.at[wid], idx_d)
  pltpu.sync_copy(ones_hbm, ones_v)
  plsc.subcore_barrier()

  def body(j, carry):
    pltpu.async_copy(table_hbm.at[idx_s.at[j]], rows, sem).wait()
    pltpu.sync_copy(rows, acc_sh.at[idx_d.at[j]], add=True)
    pltpu.sync_copy(ones_v, deg_sh.at[idx_d.at[j]], add=True)
    return carry

  lax.fori_loop(0, CH1, body, 0)
  plsc.subcore_barrier()
  pltpu.sync_copy(acc_sh.at[my], acc_out.at[cid, my])
  pltpu.sync_copy(deg_sh.at[my], deg_out.at[cid, my])


def _tc1_body(x_ref, acc_ref, deg_ref, ws_ref, wn_ref, b_ref, h_ref):
  a = acc_ref[...]
  dg = deg_ref[...]
  d = (dg[0] + dg[1])[:, 0:1]
  inv = 1.0 / jnp.maximum(d, 1.0)
  hn = (a[0] + a[1]) * inv
  h = (jnp.dot(x_ref[...], ws_ref[...], preferred_element_type=jnp.float32)
       + jnp.dot(hn, wn_ref[...], preferred_element_type=jnp.float32)
       + b_ref[...])
  h_ref[...] = jnp.maximum(h, 0.0)


def _tc2_body(h_ref, acc_ref, deg_ref, ws_ref, wn_ref, b_ref, o_ref):
  a = acc_ref[...]
  dg = deg_ref[...]
  d = (dg[0] + dg[1])[:, 0:1]
  inv = 1.0 / jnp.maximum(d, 1.0)
  hn = (a[0] + a[1]) * inv
  o_ref[...] = (jnp.dot(h_ref[...], ws_ref[...],
                        preferred_element_type=jnp.float32)
                + jnp.dot(hn, wn_ref[...], preferred_element_type=jnp.float32)
                + b_ref[...])


_BT = 1024


def kernel(x, edge_src0, edge_dst0, edge_src1, edge_dst1,
           W_self1, W_neigh1, b1, W_self2, W_neigh2, b2):
  es0 = edge_src0.reshape(NW, CH0, C0)
  ed0 = edge_dst0.reshape(NW, CH0, C0)
  es1 = edge_src1.reshape(NW, CH1, C1)
  ed1 = edge_dst1.reshape(NW, CH1, C1)

  zacc0 = jnp.zeros((N1, F), jnp.float32)
  ones0 = jnp.ones((C0, F), jnp.float32)
  acc, deg = _sc_agg0(x, es0, ed0, zacc0, ones0)

  h = pl.pallas_call(
      _tc1_body,
      grid=(N1 // _BT,),
      in_specs=[
          pl.BlockSpec((_BT, F), lambda i: (i, 0)),
          pl.BlockSpec((NC, _BT, F), lambda i: (0, i, 0)),
          pl.BlockSpec((NC, _BT, F), lambda i: (0, i, 0)),
          pl.BlockSpec((F, F), lambda i: (0, 0)),
          pl.BlockSpec((F, F), lambda i: (0, 0)),
          pl.BlockSpec((1, F), lambda i: (0, 0)),
      ],
      out_specs=pl.BlockSpec((_BT, F), lambda i: (i, 0)),
      out_shape=jax.ShapeDtypeStruct((N1, F), jnp.float32),
  )(x, acc, deg, W_self1, W_neigh1, b1.reshape(1, F))

  zacc1 = jnp.zeros((N2, F), jnp.float32)
  ones1 = jnp.ones((C1, F), jnp.float32)
  acc2, deg2 = _sc_agg1(h, es1, ed1, zacc1, ones1)

  out = pl.pallas_call(
      _tc2_body,
      grid=(1,),
      in_specs=[
          pl.BlockSpec((N2, F), lambda i: (0, 0)),
          pl.BlockSpec((NC, N2, F), lambda i: (0, 0, 0)),
          pl.BlockSpec((NC, N2, F), lambda i: (0, 0, 0)),
          pl.BlockSpec((F, C), lambda i: (0, 0)),
          pl.BlockSpec((F, C), lambda i: (0, 0)),
          pl.BlockSpec((1, C), lambda i: (0, 0)),
      ],
      out_specs=pl.BlockSpec((N2, C), lambda i: (i, 0)),
      out_shape=jax.ShapeDtypeStruct((N2, C), jnp.float32),
  )(h, acc2, deg2, W_self2, W_neigh2, b2.reshape(1, C))
  return out


# speedup vs baseline: 4.9973x; 1.1603x over previous
import functools

import jax
import jax.numpy as jnp
from jax import lax
from jax.experimental import pallas as pl
from jax.experimental.pallas import tpu as pltpu
from jax.experimental.pallas import tpu_sc as plsc

N0 = 266240
N1 = 10240
N2 = 1024
E0 = 256000
E1 = 10240
F = 128
C = 32
DW = 16

NC = 2
NS = 16
NW = NC * NS

CH0, C0 = 64, 125
CH1, C1 = 4, 80

_mesh = plsc.VectorSubcoreMesh(core_axis_name="c", subcore_axis_name="s")


@functools.partial(
    pl.kernel,
    out_type=jax.ShapeDtypeStruct((NC, N1, F), jnp.float32),
    mesh=_mesh,
    scratch_types=[
        pltpu.VMEM((CH0, C0), jnp.int32),
        pltpu.VMEM((CH0, C0), jnp.int32),
        pltpu.VMEM((2, C0, F), jnp.float32),
        pltpu.VMEM_SHARED((N1, F), jnp.float32),
        pltpu.SemaphoreType.DMA,
    ],
)
def _sc_agg0(table_hbm, es_hbm, ed_hbm, zacc_hbm,
             acc_out,
             idx_s, idx_d, rows, acc_sh, gsem):
  cid = lax.axis_index("c")
  sid = lax.axis_index("s")
  wid = sid * NC + cid
  rpt = N1 // NS
  my = pl.ds(sid * rpt, rpt)
  pltpu.sync_copy(zacc_hbm.at[my], acc_sh.at[my])
  pltpu.sync_copy(es_hbm.at[wid], idx_s)
  pltpu.sync_copy(ed_hbm.at[wid], idx_d)
  plsc.subcore_barrier()

  def body(i, carry):
    j = i - 1
    jc = jnp.clip(j, 0, CH0 - 1)
    si = lax.rem(i, 2)
    sj = lax.rem(i + 1, 2)

    @pl.when(i > 0)
    def _():
      pltpu.make_async_copy(table_hbm.at[idx_s.at[jc]], rows.at[sj],
                            gsem).wait()

    @pl.when(i < CH0)
    def _():
      pltpu.async_copy(table_hbm.at[idx_s.at[i]], rows.at[si], gsem)

    @pl.when(i > 0)
    def _():
      pltpu.sync_copy(rows.at[sj], acc_sh.at[idx_d.at[jc]], add=True)

    return carry

  lax.fori_loop(0, CH0 + 1, body, 0)
  plsc.subcore_barrier()
  pltpu.sync_copy(acc_sh.at[my], acc_out.at[cid, my])


@functools.partial(
    pl.kernel,
    out_type=jax.ShapeDtypeStruct((NC, N1, F), jnp.float32),
    mesh=_mesh,
    scratch_types=[
        pltpu.VMEM((CH0, C0), jnp.int32),
        pltpu.VMEM((C0, F), jnp.float32),
        pltpu.VMEM_SHARED((N1, F), jnp.float32),
        pltpu.SemaphoreType.DMA,
    ],
)
def _sc_deg0(ed_hbm, zacc_hbm, ones_hbm,
             deg_out,
             idx_d, ones_v, deg_sh, ssem):
  cid = lax.axis_index("c")
  sid = lax.axis_index("s")
  wid = sid * NC + cid
  rpt = N1 // NS
  my = pl.ds(sid * rpt, rpt)
  pltpu.sync_copy(zacc_hbm.at[my], deg_sh.at[my])
  pltpu.sync_copy(ed_hbm.at[wid], idx_d)
  pltpu.sync_copy(ones_hbm, ones_v)
  plsc.subcore_barrier()

  def body_deg(j, carry):
    pltpu.async_copy(ones_v, deg_sh.at[idx_d.at[j]], ssem, add=True)
    return carry

  lax.fori_loop(0, CH0, body_deg, 0)

  def drain_deg(j, carry):
    pltpu.make_async_copy(ones_v, deg_sh.at[idx_d.at[0]], ssem).wait()
    return carry

  lax.fori_loop(0, CH0, drain_deg, 0)
  plsc.subcore_barrier()
  pltpu.sync_copy(deg_sh.at[my], deg_out.at[cid, my])


@functools.partial(
    pl.kernel,
    out_type=(
        jax.ShapeDtypeStruct((NC, N2, F), jnp.float32),
        jax.ShapeDtypeStruct((NC, N2, F), jnp.float32),
    ),
    mesh=_mesh,
    scratch_types=[
        pltpu.VMEM((CH1, C1), jnp.int32),
        pltpu.VMEM((CH1, C1), jnp.int32),
        pltpu.VMEM((C1, F), jnp.float32),
        pltpu.VMEM((C1, F), jnp.float32),
        pltpu.VMEM_SHARED((N2, F), jnp.float32),
        pltpu.VMEM_SHARED((N2, F), jnp.float32),
        pltpu.SemaphoreType.DMA,
        pltpu.SemaphoreType.DMA,
        pltpu.SemaphoreType.DMA,
    ],
)
def _sc_agg1(table_hbm, es_hbm, ed_hbm, zacc_hbm, ones_hbm,
             acc_out, deg_out,
             idx_s, idx_d, rows, ones_v, acc_sh, deg_sh, gsem, ssem0, ssem1):
  cid = lax.axis_index("c")
  sid = lax.axis_index("s")
  wid = sid * NC + cid
  rpt = N2 // NS
  my = pl.ds(sid * rpt, rpt)
  pltpu.sync_copy(zacc_hbm.at[my], acc_sh.at[my])
  pltpu.sync_copy(zacc_hbm.at[my], deg_sh.at[my])
  pltpu.sync_copy(es_hbm.at[wid], idx_s)
  pltpu.sync_copy(ed_hbm.at[wid], idx_d)
  pltpu.sync_copy(ones_hbm, ones_v)
  plsc.subcore_barrier()

  def body(j, carry):
    pltpu.async_copy(table_hbm.at[idx_s.at[j]], rows, gsem).wait()
    pltpu.async_copy(rows, acc_sh.at[idx_d.at[j]], ssem0, add=True)
    pltpu.async_copy(ones_v, deg_sh.at[idx_d.at[j]], ssem1, add=True)
    pltpu.make_async_copy(rows, acc_sh.at[idx_d.at[j]], ssem0).wait()
    pltpu.make_async_copy(ones_v, deg_sh.at[idx_d.at[j]], ssem1).wait()
    return carry

  lax.fori_loop(0, CH1, body, 0)
  plsc.subcore_barrier()
  pltpu.sync_copy(acc_sh.at[my], acc_out.at[cid, my])
  pltpu.sync_copy(deg_sh.at[my], deg_out.at[cid, my])


def _tc1_body(x_ref, acc_ref, deg_ref, ws_ref, wn_ref, b_ref, h_ref):
  a = acc_ref[...]
  dg = deg_ref[...]
  d = (dg[0] + dg[1])[:, 0:1]
  inv = 1.0 / jnp.maximum(d, 1.0)
  hn = (a[0] + a[1]) * inv
  h = (jnp.dot(x_ref[...], ws_ref[...], preferred_element_type=jnp.float32)
       + jnp.dot(hn, wn_ref[...], preferred_element_type=jnp.float32)
       + b_ref[...])
  h_ref[...] = jnp.maximum(h, 0.0)


def _tc2_body(h_ref, acc_ref, deg_ref, ws_ref, wn_ref, b_ref, o_ref):
  a = acc_ref[...]
  dg = deg_ref[...]
  d = (dg[0] + dg[1])[:, 0:1]
  inv = 1.0 / jnp.maximum(d, 1.0)
  hn = (a[0] + a[1]) * inv
  o_ref[...] = (jnp.dot(h_ref[...], ws_ref[...],
                        preferred_element_type=jnp.float32)
                + jnp.dot(hn, wn_ref[...], preferred_element_type=jnp.float32)
                + b_ref[...])


_BT = 1024


def kernel(x, edge_src0, edge_dst0, edge_src1, edge_dst1,
           W_self1, W_neigh1, b1, W_self2, W_neigh2, b2):
  es0 = edge_src0.reshape(NW, CH0, C0)
  ed0 = edge_dst0.reshape(NW, CH0, C0)
  es1 = edge_src1.reshape(NW, CH1, C1)
  ed1 = edge_dst1.reshape(NW, CH1, C1)

  zacc0 = jnp.zeros((N1, F), jnp.float32)
  ones0 = jnp.ones((C0, F), jnp.float32)
  acc = _sc_agg0(x, es0, ed0, zacc0)
  deg = _sc_deg0(ed0, zacc0, ones0)

  h = pl.pallas_call(
      _tc1_body,
      grid=(N1 // _BT,),
      in_specs=[
          pl.BlockSpec((_BT, F), lambda i: (i, 0)),
          pl.BlockSpec((NC, _BT, F), lambda i: (0, i, 0)),
          pl.BlockSpec((NC, _BT, F), lambda i: (0, i, 0)),
          pl.BlockSpec((F, F), lambda i: (0, 0)),
          pl.BlockSpec((F, F), lambda i: (0, 0)),
          pl.BlockSpec((1, F), lambda i: (0, 0)),
      ],
      out_specs=pl.BlockSpec((_BT, F), lambda i: (i, 0)),
      out_shape=jax.ShapeDtypeStruct((N1, F), jnp.float32),
  )(x, acc, deg, W_self1, W_neigh1, b1.reshape(1, F))

  zacc1 = jnp.zeros((N2, F), jnp.float32)
  ones1 = jnp.ones((C1, F), jnp.float32)
  acc2, deg2 = _sc_agg1(h, es1, ed1, zacc1, ones1)

  out = pl.pallas_call(
      _tc2_body,
      grid=(1,),
      in_specs=[
          pl.BlockSpec((N2, F), lambda i: (0, 0)),
          pl.BlockSpec((NC, N2, F), lambda i: (0, 0, 0)),
          pl.BlockSpec((NC, N2, F), lambda i: (0, 0, 0)),
          pl.BlockSpec((F, C), lambda i: (0, 0)),
          pl.BlockSpec((F, C), lambda i: (0, 0)),
          pl.BlockSpec((1, C), lambda i: (0, 0)),
      ],
      out_specs=pl.BlockSpec((N2, C), lambda i: (i, 0)),
      out_shape=jax.ShapeDtypeStruct((N2, C), jnp.float32),
  )(h, acc2, deg2, W_self2, W_neigh2, b2.reshape(1, C))
  return out
